# Initial kernel scaffold; baseline (speedup 1.0000x reference)
import jax
import jax.numpy as jnp
from jax.experimental import pallas as pl


def kernel(x, edge_index, params):
    raise NotImplementedError("write your pallas kernel here")


# speedup vs baseline: 2.7773x; 2.7773x over previous
import functools

import jax
import jax.numpy as jnp
from jax import lax
from jax.experimental import pallas as pl
from jax.experimental.pallas import tpu as pltpu
from jax.experimental.pallas import tpu_sc as plsc

N = 10000
E = 160000
IN_DIM = 256
HID = 512
OUT_DIM = 128
NUM_LAYERS = 4

N_PAD = 10240
TR = 512
GRID = N_PAD // TR

NQ = 4
QD = HID // NQ
NT = 16
EPT = E // NT
CH = 80
NCH = EPT // CH
RPT = N_PAD // NT
CNT_W = 128

def _count_body(dst_hbm, cnt_hbm, idx_v, ones_v, acc_sh):
    c = lax.axis_index("c")
    s = lax.axis_index("s")

    zeros16 = jnp.zeros((16,), jnp.float32)

    def _zero_row(i, _):
        for j in range(CNT_W // 16):
            ones_v[i, pl.ds(16 * j, 16)] = zeros16
        return 0

    lax.fori_loop(0, CH, _zero_row, 0)
    for k in range(RPT // CH):
        pltpu.sync_copy(ones_v, acc_sh.at[pl.ds(s * RPT + k * CH, CH)])

    def _one_row(i, _):
        for j in range(CNT_W // 16):
            ones_v[i, pl.ds(16 * j, 16)] = zeros16 + 1.0
        return 0

    lax.fori_loop(0, CH, _one_row, 0)
    plsc.subcore_barrier()

    def _chunk(j, _):
        base = s * EPT + j * CH
        pltpu.sync_copy(dst_hbm.at[pl.ds(base, CH)], idx_v)
        pltpu.sync_copy(ones_v, acc_sh.at[idx_v], add=True)
        return 0

    lax.fori_loop(0, NCH, _chunk, 0)
    plsc.subcore_barrier()

    w = c * NT + s
    rows = N_PAD // (2 * NT)
    pltpu.sync_copy(acc_sh.at[pl.ds(w * rows, rows)],
                    cnt_hbm.at[pl.ds(w * rows, rows)])


def _agg_body(hn_hbm, src_hbm, dst_hbm, out_hbm, src_v, dst_v, rows_v,
              acc_sh, sem):
    c = lax.axis_index("c")
    s = lax.axis_index("s")
    zeros16 = jnp.zeros((16,), jnp.float32)

    for qi in range(2):
        qoff = (2 * c + qi) * N_PAD

        def _zero_row(i, _):
            for j in range(QD // 16):
                rows_v[i, pl.ds(16 * j, 16)] = zeros16
            return 0

        lax.fori_loop(0, CH, _zero_row, 0)
        for k in range(RPT // CH):
            pltpu.sync_copy(rows_v, acc_sh.at[pl.ds(s * RPT + k * CH, CH)])
        plsc.subcore_barrier()

        def _chunk(j, _):
            base = s * EPT + j * CH
            pltpu.sync_copy(src_hbm.at[pl.ds(base, CH)], src_v)
            pltpu.sync_copy(dst_hbm.at[pl.ds(base, CH)], dst_v)
            for i in range(CH // 16):
                src_v[pl.ds(16 * i, 16)] = src_v[pl.ds(16 * i, 16)] + qoff
            pltpu.async_copy(hn_hbm.at[src_v], rows_v, sem).wait()
            pltpu.sync_copy(rows_v, acc_sh.at[dst_v], add=True)
            return 0

        lax.fori_loop(0, NCH, _chunk, 0)
        plsc.subcore_barrier()

        pltpu.sync_copy(acc_sh.at[pl.ds(s * RPT, RPT)],
                        out_hbm.at[pl.ds(qoff + s * RPT, RPT)])


@functools.cache
def _sc_kernels():
    mesh = plsc.VectorSubcoreMesh(core_axis_name="c", subcore_axis_name="s")
    count_kernel = functools.partial(
        pl.kernel,
        out_type=jax.ShapeDtypeStruct((N_PAD, CNT_W), jnp.float32),
        mesh=mesh,
        scratch_types=[
            pltpu.VMEM((CH,), jnp.int32),
            pltpu.VMEM((CH, CNT_W), jnp.float32),
            pltpu.VMEM_SHARED((N_PAD, CNT_W), jnp.float32),
        ],
    )(_count_body)
    agg_kernel = functools.partial(
        pl.kernel,
        out_type=jax.ShapeDtypeStruct((NQ * N_PAD, QD), jnp.float32),
        mesh=mesh,
        scratch_types=[
            pltpu.VMEM((CH,), jnp.int32),
            pltpu.VMEM((CH,), jnp.int32),
            pltpu.VMEM((CH, QD), jnp.float32),
            pltpu.VMEM_SHARED((N_PAD, QD), jnp.float32),
            pltpu.SemaphoreType.DMA,
        ],
    )(_agg_body)
    return count_kernel, agg_kernel


def _inproj_body(x_ref, w_ref, b_ref, o_ref):
    o_ref[...] = (
        jnp.dot(x_ref[...], w_ref[...], preferred_element_type=jnp.float32)
        + b_ref[...]
    )


def _lnwr_body(h_ref, g_ref, bt_ref, wr_ref, br_ref, hn_ref, self_ref):
    h = h_ref[...]
    mu = jnp.mean(h, axis=1, keepdims=True)
    var = jnp.mean((h - mu) ** 2, axis=1, keepdims=True)
    hn = (h - mu) * lax.rsqrt(var + 1e-5) * g_ref[...] + bt_ref[...]
    hn_ref[...] = hn.reshape(TR, NQ, QD).transpose(1, 0, 2)
    self_ref[...] = (
        jnp.dot(hn, wr_ref[...], preferred_element_type=jnp.float32)
        + br_ref[...]
    )


def _mix_body(agg_ref, cnt_ref, wl_ref, bl_ref, self_ref, h_ref, jk_ref,
              ho_ref, jko_ref):
    agg = agg_ref[...].transpose(1, 0, 2).reshape(TR, HID)
    scale = 1.0 / jnp.maximum(cnt_ref[:, :1], 1.0)
    z = (
        jnp.dot(agg * scale, wl_ref[...], preferred_element_type=jnp.float32)
        + bl_ref[...]
        + self_ref[...]
    )
    hnew = jnp.maximum(h_ref[...] + z, 0.0)
    ho_ref[...] = hnew
    jko_ref[...] = jnp.maximum(jk_ref[...], hnew)


def _outproj_body(jk_ref, w_ref, b_ref, o_ref):
    o_ref[...] = (
        jnp.dot(jk_ref[...], w_ref[...], preferred_element_type=jnp.float32)
        + b_ref[...]
    )


def _row_spec(w):
    return pl.BlockSpec((TR, w), lambda i: (i, 0))


def _full_spec(shape):
    return pl.BlockSpec(shape, lambda i: tuple(0 for _ in shape))


_inproj = pl.pallas_call(
    _inproj_body,
    grid=(GRID,),
    in_specs=[_row_spec(IN_DIM), _full_spec((IN_DIM, HID)),
              _full_spec((1, HID))],
    out_specs=_row_spec(HID),
    out_shape=jax.ShapeDtypeStruct((N_PAD, HID), jnp.float32),
)

_lnwr = pl.pallas_call(
    _lnwr_body,
    grid=(GRID,),
    in_specs=[_row_spec(HID), _full_spec((1, HID)), _full_spec((1, HID)),
              _full_spec((HID, HID)), _full_spec((1, HID))],
    out_specs=[
        pl.BlockSpec((NQ, TR, QD), lambda i: (0, i, 0)),
        _row_spec(HID),
    ],
    out_shape=[
        jax.ShapeDtypeStruct((NQ, N_PAD, QD), jnp.float32),
        jax.ShapeDtypeStruct((N_PAD, HID), jnp.float32),
    ],
)

_mix = pl.pallas_call(
    _mix_body,
    grid=(GRID,),
    in_specs=[
        pl.BlockSpec((NQ, TR, QD), lambda i: (0, i, 0)),
        _row_spec(CNT_W),
        _full_spec((HID, HID)),
        _full_spec((1, HID)),
        _row_spec(HID),
        _row_spec(HID),
        _row_spec(HID),
    ],
    out_specs=[_row_spec(HID), _row_spec(HID)],
    out_shape=[
        jax.ShapeDtypeStruct((N_PAD, HID), jnp.float32),
        jax.ShapeDtypeStruct((N_PAD, HID), jnp.float32),
    ],
)

_outproj = pl.pallas_call(
    _outproj_body,
    grid=(GRID,),
    in_specs=[_row_spec(HID), _full_spec((HID, OUT_DIM)),
              _full_spec((1, OUT_DIM))],
    out_specs=_row_spec(OUT_DIM),
    out_shape=jax.ShapeDtypeStruct((N_PAD, OUT_DIM), jnp.float32),
)


def kernel(x, edge_index, params):
    src = edge_index[0].astype(jnp.int32)
    dst = edge_index[1].astype(jnp.int32)
    x_p = jnp.pad(x, ((0, N_PAD - N), (0, 0)))

    p = params
    count_kernel, agg_kernel = _sc_kernels()
    h = _inproj(x_p, p["Win"], p["bin"].reshape(1, HID))
    cnt = count_kernel(dst)

    jk = jnp.zeros((N_PAD, HID), jnp.float32)
    for i in range(NUM_LAYERS):
        hn4, self_term = _lnwr(
            h,
            p["ln_g"][i].reshape(1, HID),
            p["ln_b"][i].reshape(1, HID),
            p["Wr"][i],
            p["br"][i].reshape(1, HID),
        )
        agg = agg_kernel(hn4.reshape(NQ * N_PAD, QD), src, dst)
        h, jk = _mix(
            agg.reshape(NQ, N_PAD, QD),
            cnt,
            p["Wl"][i],
            p["bl"][i].reshape(1, HID),
            self_term,
            h,
            jk,
        )

    out = _outproj(jk, p["Wout"], p["bout"].reshape(1, OUT_DIM))
    return out[:N]


# speedup vs baseline: 5.8006x; 2.0886x over previous
import functools

import jax
import jax.numpy as jnp
from jax import lax
from jax.experimental import pallas as pl
from jax.experimental.pallas import tpu as pltpu
from jax.experimental.pallas import tpu_sc as plsc

N = 10000
E = 160000
IN_DIM = 256
HID = 512
OUT_DIM = 128
NUM_LAYERS = 4

N_PAD = 10240
TR = 512
GRID = N_PAD // TR

NQ = 4
QD = HID // NQ
NT = 16
EPT = E // NT
CH = 80
NCH = EPT // CH
RPT = N_PAD // NT
CNT_W = 128

def _count_body(dst_hbm, cnt_hbm, idx_v, ones_v, acc_sh):
    c = lax.axis_index("c")
    s = lax.axis_index("s")

    zeros16 = jnp.zeros((16,), jnp.float32)

    def _zero_row(i, _):
        for j in range(CNT_W // 16):
            ones_v[i, pl.ds(16 * j, 16)] = zeros16
        return 0

    lax.fori_loop(0, CH, _zero_row, 0)
    for k in range(RPT // CH):
        pltpu.sync_copy(ones_v, acc_sh.at[pl.ds(s * RPT + k * CH, CH)])

    def _one_row(i, _):
        for j in range(CNT_W // 16):
            ones_v[i, pl.ds(16 * j, 16)] = zeros16 + 1.0
        return 0

    lax.fori_loop(0, CH, _one_row, 0)
    plsc.subcore_barrier()

    def _chunk(j, _):
        base = s * EPT + j * CH
        pltpu.sync_copy(dst_hbm.at[pl.ds(base, CH)], idx_v)
        pltpu.sync_copy(ones_v, acc_sh.at[idx_v], add=True)
        return 0

    lax.fori_loop(0, NCH, _chunk, 0)
    plsc.subcore_barrier()

    w = c * NT + s
    rows = N_PAD // (2 * NT)
    pltpu.sync_copy(acc_sh.at[pl.ds(w * rows, rows)],
                    cnt_hbm.at[pl.ds(w * rows, rows)])


def _agg_body(hn_hbm, src_hbm, dst2_hbm, out_hbm, src_flat, dst_all,
              buf0, buf1, acc_sh, sem0, sem1):
    c = lax.axis_index("c")
    s = lax.axis_index("s")
    zeros16 = jnp.zeros((16,), jnp.float32)
    NPAIR = NCH // 2

    pltpu.sync_copy(src_hbm.at[pl.ds(s * EPT, EPT)], src_flat)
    pltpu.sync_copy(dst2_hbm.at[s], dst_all)

    def _gather(j, buf, sem):
        return pltpu.async_copy(hn_hbm.at[src_flat.at[pl.ds(j * CH, CH)]],
                                buf, sem)

    def _wait(buf, sem):
        pltpu.make_async_copy(hn_hbm.at[pl.ds(0, CH)], buf, sem).wait()

    for qi in range(2):
        qoff = (2 * c * N_PAD) if qi == 0 else N_PAD

        def _adjust(i, _):
            src_flat[pl.ds(16 * i, 16)] = src_flat[pl.ds(16 * i, 16)] + qoff
            return 0

        lax.fori_loop(0, EPT // 16, _adjust, 0)

        def _zero_row(i, _):
            for j in range(QD // 16):
                buf0[i, pl.ds(16 * j, 16)] = zeros16
            return 0

        lax.fori_loop(0, CH, _zero_row, 0)
        for k in range(RPT // CH):
            pltpu.sync_copy(buf0, acc_sh.at[pl.ds(s * RPT + k * CH, CH)])
        plsc.subcore_barrier()

        _gather(0, buf0, sem0)

        def _pair(i, _):
            _gather(2 * i + 1, buf1, sem1)
            _wait(buf0, sem0)
            pltpu.sync_copy(buf0, acc_sh.at[dst_all.at[2 * i]], add=True)
            _gather(2 * i + 2, buf0, sem0)
            _wait(buf1, sem1)
            pltpu.sync_copy(buf1, acc_sh.at[dst_all.at[2 * i + 1]], add=True)
            return 0

        lax.fori_loop(0, NPAIR, _pair, 0)
        _wait(buf0, sem0)
        pltpu.sync_copy(buf0, acc_sh.at[dst_all.at[NCH - 1]], add=True)
        plsc.subcore_barrier()

        pltpu.sync_copy(acc_sh.at[pl.ds(s * RPT, RPT)],
                        out_hbm.at[pl.ds((2 * c + qi) * N_PAD + s * RPT, RPT)])


@functools.cache
def _sc_kernels():
    mesh = plsc.VectorSubcoreMesh(core_axis_name="c", subcore_axis_name="s")
    count_kernel = functools.partial(
        pl.kernel,
        out_type=jax.ShapeDtypeStruct((N_PAD, CNT_W), jnp.float32),
        mesh=mesh,
        scratch_types=[
            pltpu.VMEM((CH,), jnp.int32),
            pltpu.VMEM((CH, CNT_W), jnp.float32),
            pltpu.VMEM_SHARED((N_PAD, CNT_W), jnp.float32),
        ],
    )(_count_body)
    agg_kernel = functools.partial(
        pl.kernel,
        out_type=jax.ShapeDtypeStruct((NQ * N_PAD, QD), jnp.float32),
        mesh=mesh,
        scratch_types=[
            pltpu.VMEM((EPT,), jnp.int32),
            pltpu.VMEM((NCH, CH), jnp.int32),
            pltpu.VMEM((CH, QD), jnp.float32),
            pltpu.VMEM((CH, QD), jnp.float32),
            pltpu.VMEM_SHARED((N_PAD, QD), jnp.float32),
            pltpu.SemaphoreType.DMA,
            pltpu.SemaphoreType.DMA,
        ],
    )(_agg_body)
    return count_kernel, agg_kernel


def _inproj_body(x_ref, w_ref, b_ref, o_ref):
    o_ref[...] = (
        jnp.dot(x_ref[...], w_ref[...], preferred_element_type=jnp.float32)
        + b_ref[...]
    )


def _lnwr_body(h_ref, g_ref, bt_ref, wr_ref, br_ref, hn_ref, self_ref):
    h = h_ref[...]
    mu = jnp.mean(h, axis=1, keepdims=True)
    var = jnp.mean((h - mu) ** 2, axis=1, keepdims=True)
    hn = (h - mu) * lax.rsqrt(var + 1e-5) * g_ref[...] + bt_ref[...]
    hn_ref[...] = hn.reshape(TR, NQ, QD).transpose(1, 0, 2)
    self_ref[...] = (
        jnp.dot(hn, wr_ref[...], preferred_element_type=jnp.float32)
        + br_ref[...]
    )


def _mix_body(agg_ref, cnt_ref, wl_ref, bl_ref, self_ref, h_ref, jk_ref,
              ho_ref, jko_ref):
    agg = agg_ref[...].transpose(1, 0, 2).reshape(TR, HID)
    scale = 1.0 / jnp.maximum(cnt_ref[:, :1], 1.0)
    z = (
        jnp.dot(agg * scale, wl_ref[...], preferred_element_type=jnp.float32)
        + bl_ref[...]
        + self_ref[...]
    )
    hnew = jnp.maximum(h_ref[...] + z, 0.0)
    ho_ref[...] = hnew
    jko_ref[...] = jnp.maximum(jk_ref[...], hnew)


def _outproj_body(jk_ref, w_ref, b_ref, o_ref):
    o_ref[...] = (
        jnp.dot(jk_ref[...], w_ref[...], preferred_element_type=jnp.float32)
        + b_ref[...]
    )


def _row_spec(w):
    return pl.BlockSpec((TR, w), lambda i: (i, 0))


def _full_spec(shape):
    return pl.BlockSpec(shape, lambda i: tuple(0 for _ in shape))


_inproj = pl.pallas_call(
    _inproj_body,
    grid=(GRID,),
    in_specs=[_row_spec(IN_DIM), _full_spec((IN_DIM, HID)),
              _full_spec((1, HID))],
    out_specs=_row_spec(HID),
    out_shape=jax.ShapeDtypeStruct((N_PAD, HID), jnp.float32),
)

_lnwr = pl.pallas_call(
    _lnwr_body,
    grid=(GRID,),
    in_specs=[_row_spec(HID), _full_spec((1, HID)), _full_spec((1, HID)),
              _full_spec((HID, HID)), _full_spec((1, HID))],
    out_specs=[
        pl.BlockSpec((NQ, TR, QD), lambda i: (0, i, 0)),
        _row_spec(HID),
    ],
    out_shape=[
        jax.ShapeDtypeStruct((NQ, N_PAD, QD), jnp.float32),
        jax.ShapeDtypeStruct((N_PAD, HID), jnp.float32),
    ],
)

_mix = pl.pallas_call(
    _mix_body,
    grid=(GRID,),
    in_specs=[
        pl.BlockSpec((NQ, TR, QD), lambda i: (0, i, 0)),
        _row_spec(CNT_W),
        _full_spec((HID, HID)),
        _full_spec((1, HID)),
        _row_spec(HID),
        _row_spec(HID),
        _row_spec(HID),
    ],
    out_specs=[_row_spec(HID), _row_spec(HID)],
    out_shape=[
        jax.ShapeDtypeStruct((N_PAD, HID), jnp.float32),
        jax.ShapeDtypeStruct((N_PAD, HID), jnp.float32),
    ],
)

_outproj = pl.pallas_call(
    _outproj_body,
    grid=(GRID,),
    in_specs=[_row_spec(HID), _full_spec((HID, OUT_DIM)),
              _full_spec((1, OUT_DIM))],
    out_specs=_row_spec(OUT_DIM),
    out_shape=jax.ShapeDtypeStruct((N_PAD, OUT_DIM), jnp.float32),
)


def kernel(x, edge_index, params):
    src = edge_index[0].astype(jnp.int32)
    dst = edge_index[1].astype(jnp.int32)
    dst3 = dst.reshape(NT, NCH, CH)
    x_p = jnp.pad(x, ((0, N_PAD - N), (0, 0)))

    p = params
    count_kernel, agg_kernel = _sc_kernels()
    h = _inproj(x_p, p["Win"], p["bin"].reshape(1, HID))
    cnt = count_kernel(dst)

    jk = jnp.zeros((N_PAD, HID), jnp.float32)
    for i in range(NUM_LAYERS):
        hn4, self_term = _lnwr(
            h,
            p["ln_g"][i].reshape(1, HID),
            p["ln_b"][i].reshape(1, HID),
            p["Wr"][i],
            p["br"][i].reshape(1, HID),
        )
        agg = agg_kernel(hn4.reshape(NQ * N_PAD, QD), src, dst3)
        h, jk = _mix(
            agg.reshape(NQ, N_PAD, QD),
            cnt,
            p["Wl"][i],
            p["bl"][i].reshape(1, HID),
            self_term,
            h,
            jk,
        )

    out = _outproj(jk, p["Wout"], p["bout"].reshape(1, OUT_DIM))
    return out[:N]
